# baseline (speedup 1.0000x reference)
import functools

import jax
import jax.numpy as jnp
from jax import lax
from jax.experimental import pallas as pl
from jax.experimental.pallas import tpu as pltpu
from jax.experimental.pallas import tpu_sc as plsc

E = 8
TOPK = 2
DIM = 2048
HIDDEN = 2048
HID2 = HIDDEN // 2
D32 = DIM // 2

BM = 256
BMS = 128
RBM = 512
NWORKERS = 32
L = 16


def _router_body(x_ref, wr_ref, sel_ref, wts_ref, rank_ref, cnt_ref,
                 base_ref, cnt_scratch):
    i = pl.program_id(0)

    @pl.when(i == 0)
    def _():
        cnt_scratch[...] = jnp.zeros_like(cnt_scratch)

    xb = x_ref[...]
    wr = wr_ref[...]
    logits = jax.lax.dot_general(
        xb, wr, (((1,), (1,)), ((), ())),
        preferred_element_type=jnp.float32,
        precision=jax.lax.Precision.DEFAULT)
    iota = jax.lax.broadcasted_iota(jnp.int32, logits.shape, 1)
    m1 = jnp.max(logits, axis=1, keepdims=True)
    i1 = jnp.min(jnp.where(logits == m1, iota, E), axis=1, keepdims=True)
    masked = jnp.where(iota == i1, -jnp.inf, logits)
    m2 = jnp.max(masked, axis=1, keepdims=True)
    i2 = jnp.min(jnp.where(masked == m2, iota, E), axis=1, keepdims=True)
    w0 = 1.0 / (1.0 + jnp.exp(m2 - m1))
    sel_ref[...] = jnp.concatenate([i1, i2], axis=1)
    wts_ref[...] = jnp.concatenate([w0, 1.0 - w0], axis=1)

    oh0 = (iota == i1).astype(jnp.float32)
    oh1 = (iota == i2).astype(jnp.float32)
    r_iota = jax.lax.broadcasted_iota(jnp.int32, (RBM, RBM), 0)
    c_iota = jax.lax.broadcasted_iota(jnp.int32, (RBM, RBM), 1)
    strict_tril = (r_iota > c_iota).astype(jnp.float32)
    excl0 = jax.lax.dot_general(
        strict_tril, oh0, (((1,), (0,)), ((), ())),
        preferred_element_type=jnp.float32)
    excl1 = jax.lax.dot_general(
        strict_tril, oh1, (((1,), (0,)), ((), ())),
        preferred_element_type=jnp.float32)
    tot0 = jnp.sum(oh0, axis=0, keepdims=True)
    tot1 = jnp.sum(oh1, axis=0, keepdims=True)
    cnt = cnt_scratch[...]
    rank0 = jnp.sum(oh0 * (excl0 + cnt), axis=1, keepdims=True)
    rank1 = jnp.sum(oh1 * (excl1 + cnt + tot0), axis=1, keepdims=True)
    rank_ref[...] = jnp.concatenate([rank0, rank1], axis=1).astype(jnp.int32)
    new_cnt = cnt + tot0 + tot1
    cnt_scratch[...] = new_cnt
    cnt_ref[...] = new_cnt.astype(jnp.int32)
    e_r = jax.lax.broadcasted_iota(jnp.int32, (E, E), 0)
    e_c = jax.lax.broadcasted_iota(jnp.int32, (E, E), 1)
    strict = (e_r < e_c).astype(jnp.float32)
    base_ref[...] = jax.lax.dot_general(
        new_cnt, strict, (((1,), (0,)), ((), ())),
        preferred_element_type=jnp.float32,
        precision=jax.lax.Precision.HIGHEST).astype(jnp.int32)


def _router(xt, w_router):
    T = xt.shape[0]
    return pl.pallas_call(
        _router_body,
        grid=(T // RBM,),
        in_specs=[
            pl.BlockSpec((RBM, DIM), lambda i: (i, 0)),
            pl.BlockSpec((E, DIM), lambda i: (0, 0)),
        ],
        out_specs=[
            pl.BlockSpec((RBM, TOPK), lambda i: (i, 0)),
            pl.BlockSpec((RBM, TOPK), lambda i: (i, 0)),
            pl.BlockSpec((RBM, TOPK), lambda i: (i, 0)),
            pl.BlockSpec((1, E), lambda i: (0, 0)),
            pl.BlockSpec((1, E), lambda i: (0, 0)),
        ],
        out_shape=[
            jax.ShapeDtypeStruct((T, TOPK), jnp.int32),
            jax.ShapeDtypeStruct((T, TOPK), jnp.float32),
            jax.ShapeDtypeStruct((T, TOPK), jnp.int32),
            jax.ShapeDtypeStruct((1, E), jnp.int32),
            jax.ShapeDtypeStruct((1, E), jnp.int32),
        ],
        scratch_shapes=[pltpu.VMEM((1, E), jnp.float32)],
    )(xt, w_router)


def _make_dispatch(T):
    t_per_w = T // NWORKERS
    n_chunks = t_per_w // L
    mesh = plsc.VectorSubcoreMesh(core_axis_name="c", subcore_axis_name="s")

    @functools.partial(
        pl.kernel, mesh=mesh,
        out_type=[
            jax.ShapeDtypeStruct((T * TOPK, D32), jnp.int32),
            jax.ShapeDtypeStruct((T,), jnp.int32),
            jax.ShapeDtypeStruct((T,), jnp.int32),
        ],
        scratch_types=[
            pltpu.VMEM((L,), jnp.int32),
            pltpu.VMEM((t_per_w,), jnp.int32),
            pltpu.VMEM((t_per_w,), jnp.int32),
            pltpu.VMEM((t_per_w,), jnp.int32),
            pltpu.VMEM((t_per_w,), jnp.int32),
            pltpu.VMEM((t_per_w,), jnp.int32),
            pltpu.VMEM((t_per_w,), jnp.int32),
            pltpu.VMEM((L, D32), jnp.int32),
            pltpu.SemaphoreType.DMA,
        ],
        compiler_params=pltpu.CompilerParams(needs_layout_passes=False),
    )
    def dispatch(xt_hbm, sel0_hbm, sel1_hbm, rank0_hbm, rank1_hbm, base_hbm,
                 xs_hbm, pos0_hbm, pos1_hbm,
                 base_v, sel0_v, sel1_v, rank0_v, rank1_v,
                 pos0_v, pos1_v, buf, sem):
        wid = lax.axis_index("s") * 2 + lax.axis_index("c")
        tbase = wid * t_per_w
        pltpu.sync_copy(base_hbm, base_v)
        pltpu.sync_copy(sel0_hbm.at[pl.ds(tbase, t_per_w)], sel0_v)
        pltpu.sync_copy(sel1_hbm.at[pl.ds(tbase, t_per_w)], sel1_v)
        pltpu.sync_copy(rank0_hbm.at[pl.ds(tbase, t_per_w)], rank0_v)
        pltpu.sync_copy(rank1_hbm.at[pl.ds(tbase, t_per_w)], rank1_v)
        for c in range(n_chunks):
            v0 = sel0_v[pl.ds(c * L, L)]
            v1 = sel1_v[pl.ds(c * L, L)]
            p0 = plsc.load_gather(base_v, [v0]) + rank0_v[pl.ds(c * L, L)]
            p1 = plsc.load_gather(base_v, [v1]) + rank1_v[pl.ds(c * L, L)]
            pos0_v[pl.ds(c * L, L)] = p0
            pos1_v[pl.ds(c * L, L)] = p1
            pltpu.sync_copy(xt_hbm.at[pl.ds(tbase + c * L, L)], buf)
            pltpu.async_copy(buf, xs_hbm.at[p0], sem).wait()
            pltpu.async_copy(buf, xs_hbm.at[p1], sem).wait()
        pltpu.sync_copy(pos0_v, pos0_hbm.at[pl.ds(tbase, t_per_w)])
        pltpu.sync_copy(pos1_v, pos1_hbm.at[pl.ds(tbase, t_per_w)])

    return dispatch


def _make_gather(T):
    t_per_w = T // NWORKERS
    n_chunks = t_per_w // L
    nbuf = 3
    mesh = plsc.VectorSubcoreMesh(core_axis_name="c", subcore_axis_name="s")

    @functools.partial(
        pl.kernel, mesh=mesh,
        out_type=[
            jax.ShapeDtypeStruct((T, D32), jnp.int32),
            jax.ShapeDtypeStruct((T, D32), jnp.int32),
        ],
        scratch_types=[
            pltpu.VMEM((t_per_w,), jnp.int32),
            pltpu.VMEM((t_per_w,), jnp.int32),
            pltpu.VMEM((L, D32), jnp.int32),
            pltpu.VMEM((L, D32), jnp.int32),
            pltpu.VMEM((L, D32), jnp.int32),
            pltpu.SemaphoreType.DMA,
            pltpu.SemaphoreType.DMA,
            pltpu.SemaphoreType.DMA,
        ],
        compiler_params=pltpu.CompilerParams(needs_layout_passes=False),
    )
    def gather(ys_hbm, pos0_hbm, pos1_hbm, g0_hbm, g1_hbm,
               pos0_v, pos1_v, bufa, bufb, bufc, sema, semb, semc):
        wid = lax.axis_index("s") * 2 + lax.axis_index("c")
        tbase = wid * t_per_w
        pltpu.sync_copy(pos0_hbm.at[pl.ds(tbase, t_per_w)], pos0_v)
        pltpu.sync_copy(pos1_hbm.at[pl.ds(tbase, t_per_w)], pos1_v)
        bufs = (bufa, bufb, bufc)
        sems = (sema, semb, semc)
        pos_vs = (pos0_v, pos1_v)
        g_hbms = (g0_hbm, g1_hbm)
        n_tr = 2 * n_chunks

        def start(i):
            c, col = divmod(i, 2)
            q = pos_vs[col][pl.ds(c * L, L)]
            return pltpu.async_copy(ys_hbm.at[q], bufs[i % nbuf],
                                    sems[i % nbuf])

        pend = [None] * n_tr
        for i in range(min(nbuf, n_tr)):
            pend[i] = start(i)
        for i in range(n_tr):
            c, col = divmod(i, 2)
            pend[i].wait()
            pltpu.sync_copy(bufs[i % nbuf],
                            g_hbms[col].at[pl.ds(tbase + c * L, L)])
            if i + nbuf < n_tr:
                pend[i + nbuf] = start(i + nbuf)

    return gather


def _gffn_half_body(has_prev, meta_ref, *refs):
    if has_prev:
        x_ref, w1_ref, w3_ref, w2_ref, prev_ref, out_ref = refs
    else:
        x_ref, w1_ref, w3_ref, w2_ref, out_ref = refs
    i = pl.program_id(0)
    first = meta_ref[2, i]
    lo = meta_ref[3, i]
    hi = meta_ref[4, i]
    m = meta_ref[1, i]

    xb = x_ref[...]
    a = jax.lax.dot_general(
        xb, w1_ref[0].astype(jnp.bfloat16), (((1,), (0,)), ((), ())),
        preferred_element_type=jnp.float32)
    b = jax.lax.dot_general(
        xb, w3_ref[0].astype(jnp.bfloat16), (((1,), (0,)), ((), ())),
        preferred_element_type=jnp.float32)
    h = (a * (1.0 / (1.0 + jnp.exp(-a))) * b).astype(jnp.bfloat16)
    y = jax.lax.dot_general(
        h, w2_ref[0].astype(jnp.bfloat16), (((1,), (0,)), ((), ())),
        preferred_element_type=jnp.float32)

    rows = m * BM + jax.lax.broadcasted_iota(jnp.int32, (BM, 1), 0)
    y = jnp.where((rows >= lo) & (rows < hi), y, 0.0)
    if has_prev:
        @pl.when(first == 1)
        def _():
            out_ref[...] = (prev_ref[...].astype(jnp.float32)
                            + y).astype(jnp.bfloat16)

        @pl.when(first == 0)
        def _():
            out_ref[...] = (out_ref[...].astype(jnp.float32)
                            + y).astype(jnp.bfloat16)
    else:
        @pl.when(first == 1)
        def _():
            out_ref[...] = y.astype(jnp.bfloat16)

        @pl.when(first == 0)
        def _():
            out_ref[...] = (out_ref[...].astype(jnp.float32)
                            + y).astype(jnp.bfloat16)


def _gffn_half(xs, w1, w3, w2, meta, n_items, nh, prev=None):
    R = xs.shape[0]
    in_specs = [
        pl.BlockSpec((BM, DIM), lambda i, meta: (meta[1, i], 0)),
        pl.BlockSpec((1, DIM, HID2), lambda i, meta: (meta[0, i], 0, nh)),
        pl.BlockSpec((1, DIM, HID2), lambda i, meta: (meta[0, i], 0, nh)),
        pl.BlockSpec((1, HID2, DIM), lambda i, meta: (meta[0, i], nh, 0)),
    ]
    args = [meta, xs, w1, w3, w2]
    kwargs = {}
    if prev is not None:
        in_specs.append(pl.BlockSpec((BM, DIM), lambda i, meta: (meta[1, i], 0)))
        args.append(prev)
        kwargs["input_output_aliases"] = {5: 0}
    grid_spec = pltpu.PrefetchScalarGridSpec(
        num_scalar_prefetch=1,
        grid=(n_items,),
        in_specs=in_specs,
        out_specs=pl.BlockSpec((BM, DIM), lambda i, meta: (meta[1, i], 0)),
    )
    return pl.pallas_call(
        functools.partial(_gffn_half_body, prev is not None),
        grid_spec=grid_spec,
        out_shape=jax.ShapeDtypeStruct((R, DIM), jnp.bfloat16),
        **kwargs,
    )(*args)


def _expert_meta(counts, n_rows, n_items):
    ends = jnp.cumsum(counts)
    starts = ends - counts
    f = starts // BM
    l = (ends - 1) // BM
    tiles = jnp.where(counts > 0, l - f + 1, 0)
    c_incl = jnp.cumsum(tiles)
    c_excl = c_incl - tiles
    n_real = c_incl[-1]
    i = jnp.arange(n_items, dtype=jnp.int32)
    e_of = jnp.sum(c_incl[None, :] <= i[:, None], axis=1)
    e_of = jnp.clip(e_of, 0, counts.shape[0] - 1).astype(jnp.int32)
    m_of = (f[e_of] + (i - c_excl[e_of])).astype(jnp.int32)
    valid = i < n_real
    last_m = (n_rows // BM) - 1
    m_of = jnp.where(valid, m_of, last_m)
    lo = jnp.where(valid, jnp.maximum(starts[e_of], m_of * BM), n_rows)
    hi = jnp.where(valid, jnp.minimum(ends[e_of], (m_of + 1) * BM), n_rows)
    first = jnp.concatenate(
        [jnp.ones((1,), jnp.int32),
         (m_of[1:] != m_of[:-1]).astype(jnp.int32)])
    first = jnp.where(valid, first, 0)
    return jnp.stack([e_of, m_of, first,
                      lo.astype(jnp.int32), hi.astype(jnp.int32)]).astype(jnp.int32)


def _shared_body(x_ref, w1_ref, w3_ref, w2_ref, out_ref):
    xb = x_ref[...]
    a = jax.lax.dot_general(
        xb, w1_ref[...], (((1,), (0,)), ((), ())),
        preferred_element_type=jnp.float32)
    b = jax.lax.dot_general(
        xb, w3_ref[...], (((1,), (0,)), ((), ())),
        preferred_element_type=jnp.float32)
    h = (a * (1.0 / (1.0 + jnp.exp(-a))) * b).astype(jnp.bfloat16)
    out_ref[...] = jax.lax.dot_general(
        h, w2_ref[...], (((1,), (0,)), ((), ())),
        preferred_element_type=jnp.float32)


def _shared_ffn(xtb, sw1b, sw3b, sw2b):
    T = xtb.shape[0]
    return pl.pallas_call(
        _shared_body,
        grid=(T // BMS,),
        in_specs=[
            pl.BlockSpec((BMS, DIM), lambda i: (i, 0)),
            pl.BlockSpec((DIM, HIDDEN), lambda i: (0, 0)),
            pl.BlockSpec((DIM, HIDDEN), lambda i: (0, 0)),
            pl.BlockSpec((HIDDEN, DIM), lambda i: (0, 0)),
        ],
        out_specs=pl.BlockSpec((BMS, DIM), lambda i: (i, 0)),
        out_shape=jax.ShapeDtypeStruct((T, DIM), jnp.float32),
    )(xtb, sw1b, sw3b, sw2b)


def _combine_body(sh_ref, g0_ref, g1_ref, w0_ref, w1_ref, out_ref):
    g0 = g0_ref[...].astype(jnp.float32)
    g1 = g1_ref[...].astype(jnp.float32)
    out_ref[...] = sh_ref[...] + w0_ref[...] * g0 + w1_ref[...] * g1


def _combine(shared, g0, g1, w0, w1):
    T = shared.shape[0]
    CB = 512
    return pl.pallas_call(
        _combine_body,
        grid=(T // CB,),
        in_specs=[
            pl.BlockSpec((CB, DIM), lambda i: (i, 0)),
            pl.BlockSpec((CB, DIM), lambda i: (i, 0)),
            pl.BlockSpec((CB, DIM), lambda i: (i, 0)),
            pl.BlockSpec((CB, 1), lambda i: (i, 0)),
            pl.BlockSpec((CB, 1), lambda i: (i, 0)),
        ],
        out_specs=pl.BlockSpec((CB, DIM), lambda i: (i, 0)),
        out_shape=jax.ShapeDtypeStruct((T, DIM), jnp.float32),
    )(shared, g0, g1, w0, w1)


def kernel(x, w_router, w1, w2, w3, sw1, sw2, sw3):
    bs, slen, dim = x.shape
    T = bs * slen
    R = T * TOPK
    xt = x.reshape(T, dim)
    xtb = xt.astype(jnp.bfloat16)
    xt_i32 = jax.lax.bitcast_convert_type(
        xtb.reshape(T, D32, 2), jnp.int32)

    sel, wts, rank, counts, base = _router(xt, w_router)

    xs_i32, pos0, pos1 = _make_dispatch(T)(
        xt_i32, sel[:, 0], sel[:, 1], rank[:, 0], rank[:, 1],
        jnp.pad(base[0], (0, L - E)))
    xs = jax.lax.bitcast_convert_type(xs_i32, jnp.bfloat16).reshape(R, DIM)

    n_items = R // BM + E - 1
    meta = _expert_meta(counts[0], R, n_items)
    part = _gffn_half(xs, w1, w3, w2, meta, n_items, 0)
    ys = _gffn_half(xs, w1, w3, w2, meta, n_items, 1, prev=part)
    ys_i32 = jax.lax.bitcast_convert_type(
        ys.reshape(R, D32, 2), jnp.int32)

    shared = _shared_ffn(xtb, sw1.astype(jnp.bfloat16),
                         sw3.astype(jnp.bfloat16), sw2.astype(jnp.bfloat16))

    g0_i32, g1_i32 = _make_gather(T)(ys_i32, pos0, pos1)
    g0 = jax.lax.bitcast_convert_type(g0_i32, jnp.bfloat16).reshape(T, DIM)
    g1 = jax.lax.bitcast_convert_type(g1_i32, jnp.bfloat16).reshape(T, DIM)
    out = _combine(shared, g0, g1, wts[:, 0:1], wts[:, 1:2])
    return out.reshape(bs, slen, dim)


# speedup vs baseline: 2.1629x; 2.1629x over previous
import functools

import jax
import jax.numpy as jnp
from jax import lax
from jax.experimental import pallas as pl
from jax.experimental.pallas import tpu as pltpu
from jax.experimental.pallas import tpu_sc as plsc

E = 8
TOPK = 2
DIM = 2048
HIDDEN = 2048
HID2 = HIDDEN // 2
D32 = DIM // 2

BM = 128
BMS = 128
RBM = 512
NWORKERS = 32
L = 16


def _router_body(x_ref, wr_ref, sel_ref, wts_ref, rank_ref, cnt_ref,
                 base_ref, cnt_scratch):
    i = pl.program_id(0)

    @pl.when(i == 0)
    def _():
        cnt_scratch[...] = jnp.zeros_like(cnt_scratch)

    xb = x_ref[...]
    wr = wr_ref[...]
    logits = jax.lax.dot_general(
        xb, wr, (((1,), (1,)), ((), ())),
        preferred_element_type=jnp.float32,
        precision=jax.lax.Precision.DEFAULT)
    iota = jax.lax.broadcasted_iota(jnp.int32, logits.shape, 1)
    m1 = jnp.max(logits, axis=1, keepdims=True)
    i1 = jnp.min(jnp.where(logits == m1, iota, E), axis=1, keepdims=True)
    masked = jnp.where(iota == i1, -jnp.inf, logits)
    m2 = jnp.max(masked, axis=1, keepdims=True)
    i2 = jnp.min(jnp.where(masked == m2, iota, E), axis=1, keepdims=True)
    w0 = 1.0 / (1.0 + jnp.exp(m2 - m1))
    sel_ref[...] = jnp.concatenate([i1, i2], axis=1)
    wts_ref[...] = jnp.concatenate([w0, 1.0 - w0], axis=1)

    oh0 = (iota == i1).astype(jnp.float32)
    oh1 = (iota == i2).astype(jnp.float32)
    r_iota = jax.lax.broadcasted_iota(jnp.int32, (RBM, RBM), 0)
    c_iota = jax.lax.broadcasted_iota(jnp.int32, (RBM, RBM), 1)
    strict_tril = (r_iota > c_iota).astype(jnp.float32)
    excl0 = jax.lax.dot_general(
        strict_tril, oh0, (((1,), (0,)), ((), ())),
        preferred_element_type=jnp.float32)
    excl1 = jax.lax.dot_general(
        strict_tril, oh1, (((1,), (0,)), ((), ())),
        preferred_element_type=jnp.float32)
    tot0 = jnp.sum(oh0, axis=0, keepdims=True)
    tot1 = jnp.sum(oh1, axis=0, keepdims=True)
    cnt = cnt_scratch[...]
    rank0 = jnp.sum(oh0 * (excl0 + cnt), axis=1, keepdims=True)
    rank1 = jnp.sum(oh1 * (excl1 + cnt + tot0), axis=1, keepdims=True)
    rank_ref[...] = jnp.concatenate([rank0, rank1], axis=1).astype(jnp.int32)
    new_cnt = cnt + tot0 + tot1
    cnt_scratch[...] = new_cnt
    cnt_ref[...] = new_cnt.astype(jnp.int32)
    e_r = jax.lax.broadcasted_iota(jnp.int32, (E, E), 0)
    e_c = jax.lax.broadcasted_iota(jnp.int32, (E, E), 1)
    strict = (e_r < e_c).astype(jnp.float32)
    base_ref[...] = jax.lax.dot_general(
        new_cnt, strict, (((1,), (0,)), ((), ())),
        preferred_element_type=jnp.float32,
        precision=jax.lax.Precision.HIGHEST).astype(jnp.int32)


def _router(xt, w_router):
    T = xt.shape[0]
    return pl.pallas_call(
        _router_body,
        grid=(T // RBM,),
        in_specs=[
            pl.BlockSpec((RBM, DIM), lambda i: (i, 0)),
            pl.BlockSpec((E, DIM), lambda i: (0, 0)),
        ],
        out_specs=[
            pl.BlockSpec((RBM, TOPK), lambda i: (i, 0)),
            pl.BlockSpec((RBM, TOPK), lambda i: (i, 0)),
            pl.BlockSpec((RBM, TOPK), lambda i: (i, 0)),
            pl.BlockSpec((1, E), lambda i: (0, 0)),
            pl.BlockSpec((1, E), lambda i: (0, 0)),
        ],
        out_shape=[
            jax.ShapeDtypeStruct((T, TOPK), jnp.int32),
            jax.ShapeDtypeStruct((T, TOPK), jnp.float32),
            jax.ShapeDtypeStruct((T, TOPK), jnp.int32),
            jax.ShapeDtypeStruct((1, E), jnp.int32),
            jax.ShapeDtypeStruct((1, E), jnp.int32),
        ],
        scratch_shapes=[pltpu.VMEM((1, E), jnp.float32)],
    )(xt, w_router)


def _make_dispatch(T):
    t_per_w = T // NWORKERS
    n_chunks = t_per_w // L
    mesh = plsc.VectorSubcoreMesh(core_axis_name="c", subcore_axis_name="s")

    @functools.partial(
        pl.kernel, mesh=mesh,
        out_type=[
            jax.ShapeDtypeStruct((T * TOPK, DIM), jnp.float32),
            jax.ShapeDtypeStruct((T,), jnp.int32),
            jax.ShapeDtypeStruct((T,), jnp.int32),
        ],
        scratch_types=[
            pltpu.VMEM((L,), jnp.int32),
            pltpu.VMEM((t_per_w,), jnp.int32),
            pltpu.VMEM((t_per_w,), jnp.int32),
            pltpu.VMEM((t_per_w,), jnp.int32),
            pltpu.VMEM((t_per_w,), jnp.int32),
            pltpu.VMEM((t_per_w,), jnp.int32),
            pltpu.VMEM((t_per_w,), jnp.int32),
            pltpu.VMEM((L, DIM), jnp.float32),
            pltpu.SemaphoreType.DMA,
        ],
        compiler_params=pltpu.CompilerParams(needs_layout_passes=False),
    )
    def dispatch(xt_hbm, sel0_hbm, sel1_hbm, rank0_hbm, rank1_hbm, base_hbm,
                 xs_hbm, pos0_hbm, pos1_hbm,
                 base_v, sel0_v, sel1_v, rank0_v, rank1_v,
                 pos0_v, pos1_v, buf, sem):
        wid = lax.axis_index("s") * 2 + lax.axis_index("c")
        tbase = wid * t_per_w
        pltpu.sync_copy(base_hbm, base_v)
        pltpu.sync_copy(sel0_hbm.at[pl.ds(tbase, t_per_w)], sel0_v)
        pltpu.sync_copy(sel1_hbm.at[pl.ds(tbase, t_per_w)], sel1_v)
        pltpu.sync_copy(rank0_hbm.at[pl.ds(tbase, t_per_w)], rank0_v)
        pltpu.sync_copy(rank1_hbm.at[pl.ds(tbase, t_per_w)], rank1_v)
        for c in range(n_chunks):
            v0 = sel0_v[pl.ds(c * L, L)]
            v1 = sel1_v[pl.ds(c * L, L)]
            p0 = plsc.load_gather(base_v, [v0]) + rank0_v[pl.ds(c * L, L)]
            p1 = plsc.load_gather(base_v, [v1]) + rank1_v[pl.ds(c * L, L)]
            pos0_v[pl.ds(c * L, L)] = p0
            pos1_v[pl.ds(c * L, L)] = p1
            pltpu.sync_copy(xt_hbm.at[pl.ds(tbase + c * L, L)], buf)
            pltpu.async_copy(buf, xs_hbm.at[p0], sem).wait()
            pltpu.async_copy(buf, xs_hbm.at[p1], sem).wait()
        pltpu.sync_copy(pos0_v, pos0_hbm.at[pl.ds(tbase, t_per_w)])
        pltpu.sync_copy(pos1_v, pos1_hbm.at[pl.ds(tbase, t_per_w)])

    return dispatch


def _make_gather(T):
    t_per_w = T // NWORKERS
    n_chunks = t_per_w // L
    nbuf = 3
    mesh = plsc.VectorSubcoreMesh(core_axis_name="c", subcore_axis_name="s")

    @functools.partial(
        pl.kernel, mesh=mesh,
        out_type=[
            jax.ShapeDtypeStruct((T, DIM), jnp.float32),
            jax.ShapeDtypeStruct((T, DIM), jnp.float32),
        ],
        scratch_types=[
            pltpu.VMEM((t_per_w,), jnp.int32),
            pltpu.VMEM((t_per_w,), jnp.int32),
            pltpu.VMEM((L, DIM), jnp.float32),
            pltpu.VMEM((L, DIM), jnp.float32),
            pltpu.VMEM((L, DIM), jnp.float32),
            pltpu.SemaphoreType.DMA,
            pltpu.SemaphoreType.DMA,
            pltpu.SemaphoreType.DMA,
        ],
        compiler_params=pltpu.CompilerParams(needs_layout_passes=False),
    )
    def gather(ys_hbm, pos0_hbm, pos1_hbm, g0_hbm, g1_hbm,
               pos0_v, pos1_v, bufa, bufb, bufc, sema, semb, semc):
        wid = lax.axis_index("s") * 2 + lax.axis_index("c")
        tbase = wid * t_per_w
        pltpu.sync_copy(pos0_hbm.at[pl.ds(tbase, t_per_w)], pos0_v)
        pltpu.sync_copy(pos1_hbm.at[pl.ds(tbase, t_per_w)], pos1_v)
        bufs = (bufa, bufb, bufc)
        sems = (sema, semb, semc)
        pos_vs = (pos0_v, pos1_v)
        g_hbms = (g0_hbm, g1_hbm)
        n_tr = 2 * n_chunks

        def start(i):
            c, col = divmod(i, 2)
            q = pos_vs[col][pl.ds(c * L, L)]
            return pltpu.async_copy(ys_hbm.at[q], bufs[i % nbuf],
                                    sems[i % nbuf])

        pend = [None] * n_tr
        for i in range(min(nbuf, n_tr)):
            pend[i] = start(i)
        for i in range(n_tr):
            c, col = divmod(i, 2)
            pend[i].wait()
            pltpu.sync_copy(bufs[i % nbuf],
                            g_hbms[col].at[pl.ds(tbase + c * L, L)])
            if i + nbuf < n_tr:
                pend[i + nbuf] = start(i + nbuf)

    return gather


def _gffn_half_body(has_prev, meta_ref, *refs):
    if has_prev:
        x_ref, w1_ref, w3_ref, w2_ref, prev_ref, out_ref = refs
    else:
        x_ref, w1_ref, w3_ref, w2_ref, out_ref = refs
    i = pl.program_id(0)
    first = meta_ref[2, i]
    lo = meta_ref[3, i]
    hi = meta_ref[4, i]
    m = meta_ref[1, i]

    xb = x_ref[...].astype(jnp.bfloat16)
    a = jax.lax.dot_general(
        xb, w1_ref[0].astype(jnp.bfloat16), (((1,), (0,)), ((), ())),
        preferred_element_type=jnp.float32)
    b = jax.lax.dot_general(
        xb, w3_ref[0].astype(jnp.bfloat16), (((1,), (0,)), ((), ())),
        preferred_element_type=jnp.float32)
    h = (a * (1.0 / (1.0 + jnp.exp(-a))) * b).astype(jnp.bfloat16)
    y = jax.lax.dot_general(
        h, w2_ref[0].astype(jnp.bfloat16), (((1,), (0,)), ((), ())),
        preferred_element_type=jnp.float32)

    rows = m * BM + jax.lax.broadcasted_iota(jnp.int32, (BM, 1), 0)
    y = jnp.where((rows >= lo) & (rows < hi), y, 0.0)
    if has_prev:
        @pl.when(first == 1)
        def _():
            out_ref[...] = prev_ref[...] + y

        @pl.when(first == 0)
        def _():
            out_ref[...] += y
    else:
        @pl.when(first == 1)
        def _():
            out_ref[...] = y

        @pl.when(first == 0)
        def _():
            out_ref[...] += y


def _gffn_half(xs, w1, w3, w2, meta, n_items, nh, prev=None):
    R = xs.shape[0]
    in_specs = [
        pl.BlockSpec((BM, DIM), lambda i, meta: (meta[1, i], 0)),
        pl.BlockSpec((1, DIM, HID2), lambda i, meta: (meta[0, i], 0, nh)),
        pl.BlockSpec((1, DIM, HID2), lambda i, meta: (meta[0, i], 0, nh)),
        pl.BlockSpec((1, HID2, DIM), lambda i, meta: (meta[0, i], nh, 0)),
    ]
    args = [meta, xs, w1, w3, w2]
    kwargs = {}
    if prev is not None:
        in_specs.append(pl.BlockSpec((BM, DIM), lambda i, meta: (meta[1, i], 0)))
        args.append(prev)
        kwargs["input_output_aliases"] = {5: 0}
    grid_spec = pltpu.PrefetchScalarGridSpec(
        num_scalar_prefetch=1,
        grid=(n_items,),
        in_specs=in_specs,
        out_specs=pl.BlockSpec((BM, DIM), lambda i, meta: (meta[1, i], 0)),
    )
    return pl.pallas_call(
        functools.partial(_gffn_half_body, prev is not None),
        grid_spec=grid_spec,
        out_shape=jax.ShapeDtypeStruct((R, DIM), jnp.float32),
        **kwargs,
    )(*args)


def _expert_meta(counts, n_rows, n_items):
    ends = jnp.cumsum(counts)
    starts = ends - counts
    f = starts // BM
    l = (ends - 1) // BM
    tiles = jnp.where(counts > 0, l - f + 1, 0)
    c_incl = jnp.cumsum(tiles)
    c_excl = c_incl - tiles
    n_real = c_incl[-1]
    i = jnp.arange(n_items, dtype=jnp.int32)
    e_of = jnp.sum(c_incl[None, :] <= i[:, None], axis=1)
    e_of = jnp.clip(e_of, 0, counts.shape[0] - 1).astype(jnp.int32)
    m_of = (f[e_of] + (i - c_excl[e_of])).astype(jnp.int32)
    valid = i < n_real
    last_m = (n_rows // BM) - 1
    m_of = jnp.where(valid, m_of, last_m)
    lo = jnp.where(valid, jnp.maximum(starts[e_of], m_of * BM), n_rows)
    hi = jnp.where(valid, jnp.minimum(ends[e_of], (m_of + 1) * BM), n_rows)
    first = jnp.concatenate(
        [jnp.ones((1,), jnp.int32),
         (m_of[1:] != m_of[:-1]).astype(jnp.int32)])
    first = jnp.where(valid, first, 0)
    return jnp.stack([e_of, m_of, first,
                      lo.astype(jnp.int32), hi.astype(jnp.int32)]).astype(jnp.int32)


def _shared_body(x_ref, w1_ref, w3_ref, w2_ref, out_ref):
    xb = x_ref[...]
    a = jax.lax.dot_general(
        xb, w1_ref[...], (((1,), (0,)), ((), ())),
        preferred_element_type=jnp.float32)
    b = jax.lax.dot_general(
        xb, w3_ref[...], (((1,), (0,)), ((), ())),
        preferred_element_type=jnp.float32)
    h = (a * (1.0 / (1.0 + jnp.exp(-a))) * b).astype(jnp.bfloat16)
    out_ref[...] = jax.lax.dot_general(
        h, w2_ref[...], (((1,), (0,)), ((), ())),
        preferred_element_type=jnp.float32)


def _shared_ffn(xtb, sw1b, sw3b, sw2b):
    T = xtb.shape[0]
    return pl.pallas_call(
        _shared_body,
        grid=(T // BMS,),
        in_specs=[
            pl.BlockSpec((BMS, DIM), lambda i: (i, 0)),
            pl.BlockSpec((DIM, HIDDEN), lambda i: (0, 0)),
            pl.BlockSpec((DIM, HIDDEN), lambda i: (0, 0)),
            pl.BlockSpec((HIDDEN, DIM), lambda i: (0, 0)),
        ],
        out_specs=pl.BlockSpec((BMS, DIM), lambda i: (i, 0)),
        out_shape=jax.ShapeDtypeStruct((T, DIM), jnp.float32),
    )(xtb, sw1b, sw3b, sw2b)


def _combine_body(sh_ref, g0_ref, g1_ref, w0_ref, w1_ref, out_ref):
    out_ref[...] = (sh_ref[...] + w0_ref[...] * g0_ref[...]
                    + w1_ref[...] * g1_ref[...])


def _combine(shared, g0, g1, w0, w1):
    T = shared.shape[0]
    CB = 512
    return pl.pallas_call(
        _combine_body,
        grid=(T // CB,),
        in_specs=[
            pl.BlockSpec((CB, DIM), lambda i: (i, 0)),
            pl.BlockSpec((CB, DIM), lambda i: (i, 0)),
            pl.BlockSpec((CB, DIM), lambda i: (i, 0)),
            pl.BlockSpec((CB, 1), lambda i: (i, 0)),
            pl.BlockSpec((CB, 1), lambda i: (i, 0)),
        ],
        out_specs=pl.BlockSpec((CB, DIM), lambda i: (i, 0)),
        out_shape=jax.ShapeDtypeStruct((T, DIM), jnp.float32),
    )(shared, g0, g1, w0, w1)


def kernel(x, w_router, w1, w2, w3, sw1, sw2, sw3):
    bs, slen, dim = x.shape
    T = bs * slen
    R = T * TOPK
    xt = x.reshape(T, dim)
    xtb = xt.astype(jnp.bfloat16)

    sel, wts, rank, counts, base = _router(xt, w_router)

    xs, pos0, pos1 = _make_dispatch(T)(
        xt, sel[:, 0], sel[:, 1], rank[:, 0], rank[:, 1],
        jnp.pad(base[0], (0, L - E)))

    n_items = R // BM + E - 1
    meta = _expert_meta(counts[0], R, n_items)
    part = _gffn_half(xs, w1, w3, w2, meta, n_items, 0)
    ys = _gffn_half(xs, w1, w3, w2, meta, n_items, 1, prev=part)

    shared = _shared_ffn(xtb, sw1.astype(jnp.bfloat16),
                         sw3.astype(jnp.bfloat16), sw2.astype(jnp.bfloat16))

    g0, g1 = _make_gather(T)(ys, pos0, pos1)
    out = _combine(shared, g0, g1, wts[:, 0:1], wts[:, 1:2])
    return out.reshape(bs, slen, dim)


# speedup vs baseline: 2.2157x; 1.0244x over previous
import functools

import jax
import jax.numpy as jnp
from jax import lax
from jax.experimental import pallas as pl
from jax.experimental.pallas import tpu as pltpu
from jax.experimental.pallas import tpu_sc as plsc

E = 8
TOPK = 2
DIM = 2048
HIDDEN = 2048
HID2 = HIDDEN // 2
D32 = DIM // 2

BM = 128
BMS = 128
RBM = 512
NWORKERS = 32
L = 16


def _router_body(x_ref, wr_ref, sel_ref, wts_ref, rank_ref, cnt_ref,
                 base_ref, xtb_ref, cnt_scratch):
    xtb_ref[...] = x_ref[...].astype(jnp.bfloat16)
    i = pl.program_id(0)

    @pl.when(i == 0)
    def _():
        cnt_scratch[...] = jnp.zeros_like(cnt_scratch)

    xb = x_ref[...]
    wr = wr_ref[...]
    logits = jax.lax.dot_general(
        xb, wr, (((1,), (1,)), ((), ())),
        preferred_element_type=jnp.float32,
        precision=jax.lax.Precision.DEFAULT)
    iota = jax.lax.broadcasted_iota(jnp.int32, logits.shape, 1)
    m1 = jnp.max(logits, axis=1, keepdims=True)
    i1 = jnp.min(jnp.where(logits == m1, iota, E), axis=1, keepdims=True)
    masked = jnp.where(iota == i1, -jnp.inf, logits)
    m2 = jnp.max(masked, axis=1, keepdims=True)
    i2 = jnp.min(jnp.where(masked == m2, iota, E), axis=1, keepdims=True)
    w0 = 1.0 / (1.0 + jnp.exp(m2 - m1))
    sel_ref[...] = jnp.concatenate([i1, i2], axis=1)
    wts_ref[...] = jnp.concatenate([w0, 1.0 - w0], axis=1)

    oh0 = (iota == i1).astype(jnp.float32)
    oh1 = (iota == i2).astype(jnp.float32)
    r_iota = jax.lax.broadcasted_iota(jnp.int32, (RBM, RBM), 0)
    c_iota = jax.lax.broadcasted_iota(jnp.int32, (RBM, RBM), 1)
    strict_tril = (r_iota > c_iota).astype(jnp.float32)
    excl0 = jax.lax.dot_general(
        strict_tril, oh0, (((1,), (0,)), ((), ())),
        preferred_element_type=jnp.float32)
    excl1 = jax.lax.dot_general(
        strict_tril, oh1, (((1,), (0,)), ((), ())),
        preferred_element_type=jnp.float32)
    tot0 = jnp.sum(oh0, axis=0, keepdims=True)
    tot1 = jnp.sum(oh1, axis=0, keepdims=True)
    cnt = cnt_scratch[...]
    rank0 = jnp.sum(oh0 * (excl0 + cnt), axis=1, keepdims=True)
    rank1 = jnp.sum(oh1 * (excl1 + cnt + tot0), axis=1, keepdims=True)
    rank_ref[...] = jnp.concatenate([rank0, rank1], axis=1).astype(jnp.int32)
    new_cnt = cnt + tot0 + tot1
    cnt_scratch[...] = new_cnt
    cnt_ref[...] = new_cnt.astype(jnp.int32)
    e_r = jax.lax.broadcasted_iota(jnp.int32, (E, E), 0)
    e_c = jax.lax.broadcasted_iota(jnp.int32, (E, E), 1)
    strict = (e_r < e_c).astype(jnp.float32)
    base_ref[...] = jax.lax.dot_general(
        new_cnt, strict, (((1,), (0,)), ((), ())),
        preferred_element_type=jnp.float32,
        precision=jax.lax.Precision.HIGHEST).astype(jnp.int32)


def _router(xt, w_router):
    T = xt.shape[0]
    return pl.pallas_call(
        _router_body,
        grid=(T // RBM,),
        in_specs=[
            pl.BlockSpec((RBM, DIM), lambda i: (i, 0)),
            pl.BlockSpec((E, DIM), lambda i: (0, 0)),
        ],
        out_specs=[
            pl.BlockSpec((RBM, TOPK), lambda i: (i, 0)),
            pl.BlockSpec((RBM, TOPK), lambda i: (i, 0)),
            pl.BlockSpec((RBM, TOPK), lambda i: (i, 0)),
            pl.BlockSpec((1, E), lambda i: (0, 0)),
            pl.BlockSpec((1, E), lambda i: (0, 0)),
            pl.BlockSpec((RBM, DIM), lambda i: (i, 0)),
        ],
        out_shape=[
            jax.ShapeDtypeStruct((T, TOPK), jnp.int32),
            jax.ShapeDtypeStruct((T, TOPK), jnp.float32),
            jax.ShapeDtypeStruct((T, TOPK), jnp.int32),
            jax.ShapeDtypeStruct((1, E), jnp.int32),
            jax.ShapeDtypeStruct((1, E), jnp.int32),
            jax.ShapeDtypeStruct((T, DIM), jnp.bfloat16),
        ],
        scratch_shapes=[pltpu.VMEM((1, E), jnp.float32)],
    )(xt, w_router)


def _make_dispatch(T):
    t_per_w = T // NWORKERS
    n_chunks = t_per_w // L
    mesh = plsc.VectorSubcoreMesh(core_axis_name="c", subcore_axis_name="s")

    @functools.partial(
        pl.kernel, mesh=mesh,
        out_type=[
            jax.ShapeDtypeStruct((T * TOPK, DIM), jnp.float32),
            jax.ShapeDtypeStruct((T,), jnp.int32),
            jax.ShapeDtypeStruct((T,), jnp.int32),
        ],
        scratch_types=[
            pltpu.VMEM((L,), jnp.int32),
            pltpu.VMEM((t_per_w,), jnp.int32),
            pltpu.VMEM((t_per_w,), jnp.int32),
            pltpu.VMEM((t_per_w,), jnp.int32),
            pltpu.VMEM((t_per_w,), jnp.int32),
            pltpu.VMEM((t_per_w,), jnp.int32),
            pltpu.VMEM((t_per_w,), jnp.int32),
            pltpu.VMEM((L, DIM), jnp.float32),
            pltpu.SemaphoreType.DMA,
        ],
        compiler_params=pltpu.CompilerParams(needs_layout_passes=False),
    )
    def dispatch(xt_hbm, sel0_hbm, sel1_hbm, rank0_hbm, rank1_hbm, base_hbm,
                 xs_hbm, pos0_hbm, pos1_hbm,
                 base_v, sel0_v, sel1_v, rank0_v, rank1_v,
                 pos0_v, pos1_v, buf, sem):
        wid = lax.axis_index("s") * 2 + lax.axis_index("c")
        tbase = wid * t_per_w
        pltpu.sync_copy(base_hbm, base_v)
        pltpu.sync_copy(sel0_hbm.at[pl.ds(tbase, t_per_w)], sel0_v)
        pltpu.sync_copy(sel1_hbm.at[pl.ds(tbase, t_per_w)], sel1_v)
        pltpu.sync_copy(rank0_hbm.at[pl.ds(tbase, t_per_w)], rank0_v)
        pltpu.sync_copy(rank1_hbm.at[pl.ds(tbase, t_per_w)], rank1_v)
        for c in range(n_chunks):
            v0 = sel0_v[pl.ds(c * L, L)]
            v1 = sel1_v[pl.ds(c * L, L)]
            p0 = plsc.load_gather(base_v, [v0]) + rank0_v[pl.ds(c * L, L)]
            p1 = plsc.load_gather(base_v, [v1]) + rank1_v[pl.ds(c * L, L)]
            pos0_v[pl.ds(c * L, L)] = p0
            pos1_v[pl.ds(c * L, L)] = p1
            pltpu.sync_copy(xt_hbm.at[pl.ds(tbase + c * L, L)], buf)
            pltpu.async_copy(buf, xs_hbm.at[p0], sem).wait()
            pltpu.async_copy(buf, xs_hbm.at[p1], sem).wait()
        pltpu.sync_copy(pos0_v, pos0_hbm.at[pl.ds(tbase, t_per_w)])
        pltpu.sync_copy(pos1_v, pos1_hbm.at[pl.ds(tbase, t_per_w)])

    return dispatch


def _make_gather(T):
    t_per_w = T // NWORKERS
    n_chunks = t_per_w // L
    nbuf = 3
    mesh = plsc.VectorSubcoreMesh(core_axis_name="c", subcore_axis_name="s")

    @functools.partial(
        pl.kernel, mesh=mesh,
        out_type=[
            jax.ShapeDtypeStruct((T, DIM), jnp.float32),
            jax.ShapeDtypeStruct((T, DIM), jnp.float32),
        ],
        scratch_types=[
            pltpu.VMEM((t_per_w,), jnp.int32),
            pltpu.VMEM((t_per_w,), jnp.int32),
            pltpu.VMEM((L, DIM), jnp.float32),
            pltpu.VMEM((L, DIM), jnp.float32),
            pltpu.VMEM((L, DIM), jnp.float32),
            pltpu.SemaphoreType.DMA,
            pltpu.SemaphoreType.DMA,
            pltpu.SemaphoreType.DMA,
        ],
        compiler_params=pltpu.CompilerParams(needs_layout_passes=False),
    )
    def gather(ys_hbm, pos0_hbm, pos1_hbm, g0_hbm, g1_hbm,
               pos0_v, pos1_v, bufa, bufb, bufc, sema, semb, semc):
        wid = lax.axis_index("s") * 2 + lax.axis_index("c")
        tbase = wid * t_per_w
        pltpu.sync_copy(pos0_hbm.at[pl.ds(tbase, t_per_w)], pos0_v)
        pltpu.sync_copy(pos1_hbm.at[pl.ds(tbase, t_per_w)], pos1_v)
        bufs = (bufa, bufb, bufc)
        sems = (sema, semb, semc)
        pos_vs = (pos0_v, pos1_v)
        g_hbms = (g0_hbm, g1_hbm)
        n_tr = 2 * n_chunks

        def start(i):
            c, col = divmod(i, 2)
            q = pos_vs[col][pl.ds(c * L, L)]
            return pltpu.async_copy(ys_hbm.at[q], bufs[i % nbuf],
                                    sems[i % nbuf])

        pend = [None] * n_tr
        for i in range(min(nbuf, n_tr)):
            pend[i] = start(i)
        for i in range(n_tr):
            c, col = divmod(i, 2)
            pend[i].wait()
            pltpu.sync_copy(bufs[i % nbuf],
                            g_hbms[col].at[pl.ds(tbase + c * L, L)])
            if i + nbuf < n_tr:
                pend[i + nbuf] = start(i + nbuf)

    return gather


def _gffn_half_body(has_prev, meta_ref, *refs):
    if has_prev:
        x_ref, w1_ref, w3_ref, w2_ref, prev_ref, out_ref = refs
    else:
        x_ref, w1_ref, w3_ref, w2_ref, out_ref = refs
    i = pl.program_id(0)
    first = meta_ref[2, i]
    lo = meta_ref[3, i]
    hi = meta_ref[4, i]
    m = meta_ref[1, i]

    xb = x_ref[...].astype(jnp.bfloat16)
    a = jax.lax.dot_general(
        xb, w1_ref[0].astype(jnp.bfloat16), (((1,), (0,)), ((), ())),
        preferred_element_type=jnp.float32)
    b = jax.lax.dot_general(
        xb, w3_ref[0].astype(jnp.bfloat16), (((1,), (0,)), ((), ())),
        preferred_element_type=jnp.float32)
    h = (a * (1.0 / (1.0 + jnp.exp(-a))) * b).astype(jnp.bfloat16)
    y = jax.lax.dot_general(
        h, w2_ref[0].astype(jnp.bfloat16), (((1,), (0,)), ((), ())),
        preferred_element_type=jnp.float32)

    rows = m * BM + jax.lax.broadcasted_iota(jnp.int32, (BM, 1), 0)
    y = jnp.where((rows >= lo) & (rows < hi), y, 0.0)
    if has_prev:
        @pl.when(first == 1)
        def _():
            out_ref[...] = prev_ref[...] + y

        @pl.when(first == 0)
        def _():
            out_ref[...] += y
    else:
        @pl.when(first == 1)
        def _():
            out_ref[...] = y

        @pl.when(first == 0)
        def _():
            out_ref[...] += y


def _gffn_half(xs, w1, w3, w2, meta, n_items, nh, prev=None):
    R = xs.shape[0]
    in_specs = [
        pl.BlockSpec((BM, DIM), lambda i, meta: (meta[1, i], 0)),
        pl.BlockSpec((1, DIM, HID2), lambda i, meta: (meta[0, i], 0, nh)),
        pl.BlockSpec((1, DIM, HID2), lambda i, meta: (meta[0, i], 0, nh)),
        pl.BlockSpec((1, HID2, DIM), lambda i, meta: (meta[0, i], nh, 0)),
    ]
    args = [meta, xs, w1, w3, w2]
    kwargs = {}
    if prev is not None:
        in_specs.append(pl.BlockSpec((BM, DIM), lambda i, meta: (meta[1, i], 0)))
        args.append(prev)
        kwargs["input_output_aliases"] = {5: 0}
    grid_spec = pltpu.PrefetchScalarGridSpec(
        num_scalar_prefetch=1,
        grid=(n_items,),
        in_specs=in_specs,
        out_specs=pl.BlockSpec((BM, DIM), lambda i, meta: (meta[1, i], 0)),
    )
    return pl.pallas_call(
        functools.partial(_gffn_half_body, prev is not None),
        grid_spec=grid_spec,
        out_shape=jax.ShapeDtypeStruct((R, DIM), jnp.float32),
        **kwargs,
    )(*args)


def _expert_meta(counts, n_rows, n_items):
    ends = jnp.cumsum(counts)
    starts = ends - counts
    f = starts // BM
    l = (ends - 1) // BM
    tiles = jnp.where(counts > 0, l - f + 1, 0)
    c_incl = jnp.cumsum(tiles)
    c_excl = c_incl - tiles
    n_real = c_incl[-1]
    i = jnp.arange(n_items, dtype=jnp.int32)
    e_of = jnp.sum(c_incl[None, :] <= i[:, None], axis=1)
    e_of = jnp.clip(e_of, 0, counts.shape[0] - 1).astype(jnp.int32)
    m_of = (f[e_of] + (i - c_excl[e_of])).astype(jnp.int32)
    valid = i < n_real
    last_m = (n_rows // BM) - 1
    m_of = jnp.where(valid, m_of, last_m)
    lo = jnp.where(valid, jnp.maximum(starts[e_of], m_of * BM), n_rows)
    hi = jnp.where(valid, jnp.minimum(ends[e_of], (m_of + 1) * BM), n_rows)
    first = jnp.concatenate(
        [jnp.ones((1,), jnp.int32),
         (m_of[1:] != m_of[:-1]).astype(jnp.int32)])
    first = jnp.where(valid, first, 0)
    return jnp.stack([e_of, m_of, first,
                      lo.astype(jnp.int32), hi.astype(jnp.int32)]).astype(jnp.int32)


def _shared_body(x_ref, w1_ref, w3_ref, w2_ref, out_ref):
    xb = x_ref[...]
    a = jax.lax.dot_general(
        xb, w1_ref[...], (((1,), (0,)), ((), ())),
        preferred_element_type=jnp.float32)
    b = jax.lax.dot_general(
        xb, w3_ref[...], (((1,), (0,)), ((), ())),
        preferred_element_type=jnp.float32)
    h = (a * (1.0 / (1.0 + jnp.exp(-a))) * b).astype(jnp.bfloat16)
    out_ref[...] = jax.lax.dot_general(
        h, w2_ref[...], (((1,), (0,)), ((), ())),
        preferred_element_type=jnp.float32)


def _shared_ffn(xtb, sw1b, sw3b, sw2b):
    T = xtb.shape[0]
    return pl.pallas_call(
        _shared_body,
        grid=(T // BMS,),
        in_specs=[
            pl.BlockSpec((BMS, DIM), lambda i: (i, 0)),
            pl.BlockSpec((DIM, HIDDEN), lambda i: (0, 0)),
            pl.BlockSpec((DIM, HIDDEN), lambda i: (0, 0)),
            pl.BlockSpec((HIDDEN, DIM), lambda i: (0, 0)),
        ],
        out_specs=pl.BlockSpec((BMS, DIM), lambda i: (i, 0)),
        out_shape=jax.ShapeDtypeStruct((T, DIM), jnp.float32),
    )(xtb, sw1b, sw3b, sw2b)


def _combine_body(has_prev, *refs):
    if has_prev:
        sh_ref, g0_ref, g1_ref, w0_ref, w1_ref, _prev, out_ref = refs
    else:
        sh_ref, g0_ref, g1_ref, w0_ref, w1_ref, out_ref = refs
    out_ref[...] = (sh_ref[...] + w0_ref[...] * g0_ref[...]
                    + w1_ref[...] * g1_ref[...])


def _combine_half(shared, g0, g1, w0, w1, half, prev=None):
    T = shared.shape[0]
    T2 = T // 2
    CB = 512
    nb = T2 // CB
    offb = half * nb
    in_specs = [
        pl.BlockSpec((CB, DIM), lambda i, offb=offb: (i + offb, 0)),
        pl.BlockSpec((CB, DIM), lambda i: (i, 0)),
        pl.BlockSpec((CB, DIM), lambda i: (i, 0)),
        pl.BlockSpec((CB, 1), lambda i, offb=offb: (i + offb, 0)),
        pl.BlockSpec((CB, 1), lambda i, offb=offb: (i + offb, 0)),
    ]
    args = [shared, g0, g1, w0, w1]
    kwargs = {}
    if prev is not None:
        in_specs.append(
            pl.BlockSpec((CB, DIM), lambda i, offb=offb: (i + offb, 0)))
        args.append(prev)
        kwargs["input_output_aliases"] = {5: 0}
    return pl.pallas_call(
        functools.partial(_combine_body, prev is not None),
        grid=(nb,),
        in_specs=in_specs,
        out_specs=pl.BlockSpec((CB, DIM), lambda i, offb=offb: (i + offb, 0)),
        out_shape=jax.ShapeDtypeStruct((T, DIM), jnp.float32),
        **kwargs,
    )(*args)


def kernel(x, w_router, w1, w2, w3, sw1, sw2, sw3):
    bs, slen, dim = x.shape
    T = bs * slen
    R = T * TOPK
    xt = x.reshape(T, dim)

    sel, wts, rank, counts, base, xtb = _router(xt, w_router)

    xs, pos0, pos1 = _make_dispatch(T)(
        xt, sel[:, 0], sel[:, 1], rank[:, 0], rank[:, 1],
        jnp.pad(base[0], (0, L - E)))

    n_items = R // BM + E - 1
    meta = _expert_meta(counts[0], R, n_items)
    part = _gffn_half(xs, w1, w3, w2, meta, n_items, 0)
    ys = _gffn_half(xs, w1, w3, w2, meta, n_items, 1, prev=part)

    shared = _shared_ffn(xtb, sw1.astype(jnp.bfloat16),
                         sw3.astype(jnp.bfloat16), sw2.astype(jnp.bfloat16))

    T2 = T // 2
    gat = _make_gather(T2)
    g0a, g1a = gat(ys, pos0[:T2], pos1[:T2])
    g0b, g1b = gat(ys, pos0[T2:], pos1[T2:])
    w0 = wts[:, 0:1]
    w1 = wts[:, 1:2]
    out_a = _combine_half(shared, g0a, g1a, w0, w1, 0)
    out = _combine_half(shared, g0b, g1b, w0, w1, 1, prev=out_a)
    return out.reshape(bs, slen, dim)


# speedup vs baseline: 2.2307x; 1.0067x over previous
import functools

import jax
import jax.numpy as jnp
from jax import lax
from jax.experimental import pallas as pl
from jax.experimental.pallas import tpu as pltpu
from jax.experimental.pallas import tpu_sc as plsc

E = 8
TOPK = 2
DIM = 2048
HIDDEN = 2048
HID2 = HIDDEN // 2
D32 = DIM // 2

BM = 128
BMS = 256
RBM = 512
NWORKERS = 32
L = 16


def _router_body(x_ref, wr_ref, sel_ref, wts_ref, rank_ref, cnt_ref,
                 base_ref, xtb_ref, cnt_scratch):
    xtb_ref[...] = x_ref[...].astype(jnp.bfloat16)
    i = pl.program_id(0)

    @pl.when(i == 0)
    def _():
        cnt_scratch[...] = jnp.zeros_like(cnt_scratch)

    xb = x_ref[...]
    wr = wr_ref[...]
    logits = jax.lax.dot_general(
        xb, wr, (((1,), (1,)), ((), ())),
        preferred_element_type=jnp.float32,
        precision=jax.lax.Precision.DEFAULT)
    iota = jax.lax.broadcasted_iota(jnp.int32, logits.shape, 1)
    m1 = jnp.max(logits, axis=1, keepdims=True)
    i1 = jnp.min(jnp.where(logits == m1, iota, E), axis=1, keepdims=True)
    masked = jnp.where(iota == i1, -jnp.inf, logits)
    m2 = jnp.max(masked, axis=1, keepdims=True)
    i2 = jnp.min(jnp.where(masked == m2, iota, E), axis=1, keepdims=True)
    w0 = 1.0 / (1.0 + jnp.exp(m2 - m1))
    sel_ref[...] = jnp.concatenate([i1, i2], axis=1)
    wts_ref[...] = jnp.concatenate([w0, 1.0 - w0], axis=1)

    oh0 = (iota == i1).astype(jnp.float32)
    oh1 = (iota == i2).astype(jnp.float32)
    r_iota = jax.lax.broadcasted_iota(jnp.int32, (RBM, RBM), 0)
    c_iota = jax.lax.broadcasted_iota(jnp.int32, (RBM, RBM), 1)
    strict_tril = (r_iota > c_iota).astype(jnp.float32)
    excl0 = jax.lax.dot_general(
        strict_tril, oh0, (((1,), (0,)), ((), ())),
        preferred_element_type=jnp.float32)
    excl1 = jax.lax.dot_general(
        strict_tril, oh1, (((1,), (0,)), ((), ())),
        preferred_element_type=jnp.float32)
    tot0 = jnp.sum(oh0, axis=0, keepdims=True)
    tot1 = jnp.sum(oh1, axis=0, keepdims=True)
    cnt = cnt_scratch[...]
    rank0 = jnp.sum(oh0 * (excl0 + cnt), axis=1, keepdims=True)
    rank1 = jnp.sum(oh1 * (excl1 + cnt + tot0), axis=1, keepdims=True)
    rank_ref[...] = jnp.concatenate([rank0, rank1], axis=1).astype(jnp.int32)
    new_cnt = cnt + tot0 + tot1
    cnt_scratch[...] = new_cnt
    cnt_ref[...] = new_cnt.astype(jnp.int32)
    e_r = jax.lax.broadcasted_iota(jnp.int32, (E, E), 0)
    e_c = jax.lax.broadcasted_iota(jnp.int32, (E, E), 1)
    strict = (e_r < e_c).astype(jnp.float32)
    base_ref[...] = jax.lax.dot_general(
        new_cnt, strict, (((1,), (0,)), ((), ())),
        preferred_element_type=jnp.float32,
        precision=jax.lax.Precision.HIGHEST).astype(jnp.int32)


def _router(xt, w_router):
    T = xt.shape[0]
    return pl.pallas_call(
        _router_body,
        grid=(T // RBM,),
        in_specs=[
            pl.BlockSpec((RBM, DIM), lambda i: (i, 0)),
            pl.BlockSpec((E, DIM), lambda i: (0, 0)),
        ],
        out_specs=[
            pl.BlockSpec((RBM, TOPK), lambda i: (i, 0)),
            pl.BlockSpec((RBM, TOPK), lambda i: (i, 0)),
            pl.BlockSpec((RBM, TOPK), lambda i: (i, 0)),
            pl.BlockSpec((1, E), lambda i: (0, 0)),
            pl.BlockSpec((1, E), lambda i: (0, 0)),
            pl.BlockSpec((RBM, DIM), lambda i: (i, 0)),
        ],
        out_shape=[
            jax.ShapeDtypeStruct((T, TOPK), jnp.int32),
            jax.ShapeDtypeStruct((T, TOPK), jnp.float32),
            jax.ShapeDtypeStruct((T, TOPK), jnp.int32),
            jax.ShapeDtypeStruct((1, E), jnp.int32),
            jax.ShapeDtypeStruct((1, E), jnp.int32),
            jax.ShapeDtypeStruct((T, DIM), jnp.bfloat16),
        ],
        scratch_shapes=[pltpu.VMEM((1, E), jnp.float32)],
    )(xt, w_router)


def _make_dispatch(T):
    t_per_w = T // NWORKERS
    n_chunks = t_per_w // L
    mesh = plsc.VectorSubcoreMesh(core_axis_name="c", subcore_axis_name="s")

    @functools.partial(
        pl.kernel, mesh=mesh,
        out_type=[
            jax.ShapeDtypeStruct((T * TOPK, DIM), jnp.float32),
            jax.ShapeDtypeStruct((T,), jnp.int32),
            jax.ShapeDtypeStruct((T,), jnp.int32),
        ],
        scratch_types=[
            pltpu.VMEM((L,), jnp.int32),
            pltpu.VMEM((t_per_w,), jnp.int32),
            pltpu.VMEM((t_per_w,), jnp.int32),
            pltpu.VMEM((t_per_w,), jnp.int32),
            pltpu.VMEM((t_per_w,), jnp.int32),
            pltpu.VMEM((t_per_w,), jnp.int32),
            pltpu.VMEM((t_per_w,), jnp.int32),
            pltpu.VMEM((L, DIM), jnp.float32),
            pltpu.SemaphoreType.DMA,
        ],
        compiler_params=pltpu.CompilerParams(needs_layout_passes=False),
    )
    def dispatch(xt_hbm, sel0_hbm, sel1_hbm, rank0_hbm, rank1_hbm, base_hbm,
                 xs_hbm, pos0_hbm, pos1_hbm,
                 base_v, sel0_v, sel1_v, rank0_v, rank1_v,
                 pos0_v, pos1_v, buf, sem):
        wid = lax.axis_index("s") * 2 + lax.axis_index("c")
        tbase = wid * t_per_w
        pltpu.sync_copy(base_hbm, base_v)
        pltpu.sync_copy(sel0_hbm.at[pl.ds(tbase, t_per_w)], sel0_v)
        pltpu.sync_copy(sel1_hbm.at[pl.ds(tbase, t_per_w)], sel1_v)
        pltpu.sync_copy(rank0_hbm.at[pl.ds(tbase, t_per_w)], rank0_v)
        pltpu.sync_copy(rank1_hbm.at[pl.ds(tbase, t_per_w)], rank1_v)
        for c in range(n_chunks):
            v0 = sel0_v[pl.ds(c * L, L)]
            v1 = sel1_v[pl.ds(c * L, L)]
            p0 = plsc.load_gather(base_v, [v0]) + rank0_v[pl.ds(c * L, L)]
            p1 = plsc.load_gather(base_v, [v1]) + rank1_v[pl.ds(c * L, L)]
            pos0_v[pl.ds(c * L, L)] = p0
            pos1_v[pl.ds(c * L, L)] = p1
            pltpu.sync_copy(xt_hbm.at[pl.ds(tbase + c * L, L)], buf)
            pltpu.async_copy(buf, xs_hbm.at[p0], sem).wait()
            pltpu.async_copy(buf, xs_hbm.at[p1], sem).wait()
        pltpu.sync_copy(pos0_v, pos0_hbm.at[pl.ds(tbase, t_per_w)])
        pltpu.sync_copy(pos1_v, pos1_hbm.at[pl.ds(tbase, t_per_w)])

    return dispatch


def _make_gather(T):
    t_per_w = T // NWORKERS
    n_chunks = t_per_w // L
    nbuf = 3
    mesh = plsc.VectorSubcoreMesh(core_axis_name="c", subcore_axis_name="s")

    @functools.partial(
        pl.kernel, mesh=mesh,
        out_type=[
            jax.ShapeDtypeStruct((T, DIM), jnp.float32),
            jax.ShapeDtypeStruct((T, DIM), jnp.float32),
        ],
        scratch_types=[
            pltpu.VMEM((t_per_w,), jnp.int32),
            pltpu.VMEM((t_per_w,), jnp.int32),
            pltpu.VMEM((L, DIM), jnp.float32),
            pltpu.VMEM((L, DIM), jnp.float32),
            pltpu.VMEM((L, DIM), jnp.float32),
            pltpu.SemaphoreType.DMA,
            pltpu.SemaphoreType.DMA,
            pltpu.SemaphoreType.DMA,
        ],
        compiler_params=pltpu.CompilerParams(needs_layout_passes=False),
    )
    def gather(ys_hbm, pos0_hbm, pos1_hbm, g0_hbm, g1_hbm,
               pos0_v, pos1_v, bufa, bufb, bufc, sema, semb, semc):
        wid = lax.axis_index("s") * 2 + lax.axis_index("c")
        tbase = wid * t_per_w
        pltpu.sync_copy(pos0_hbm.at[pl.ds(tbase, t_per_w)], pos0_v)
        pltpu.sync_copy(pos1_hbm.at[pl.ds(tbase, t_per_w)], pos1_v)
        bufs = (bufa, bufb, bufc)
        sems = (sema, semb, semc)
        pos_vs = (pos0_v, pos1_v)
        g_hbms = (g0_hbm, g1_hbm)
        n_tr = 2 * n_chunks

        def start(i):
            c, col = divmod(i, 2)
            q = pos_vs[col][pl.ds(c * L, L)]
            return pltpu.async_copy(ys_hbm.at[q], bufs[i % nbuf],
                                    sems[i % nbuf])

        pend = [None] * n_tr
        for i in range(min(nbuf, n_tr)):
            pend[i] = start(i)
        for i in range(n_tr):
            c, col = divmod(i, 2)
            pend[i].wait()
            pltpu.sync_copy(bufs[i % nbuf],
                            g_hbms[col].at[pl.ds(tbase + c * L, L)])
            if i + nbuf < n_tr:
                pend[i + nbuf] = start(i + nbuf)

    return gather


def _gffn_half_body(has_prev, meta_ref, *refs):
    if has_prev:
        x_ref, w1_ref, w3_ref, w2_ref, prev_ref, out_ref = refs
    else:
        x_ref, w1_ref, w3_ref, w2_ref, out_ref = refs
    i = pl.program_id(0)
    first = meta_ref[2, i]
    lo = meta_ref[3, i]
    hi = meta_ref[4, i]
    m = meta_ref[1, i]

    xb = x_ref[...].astype(jnp.bfloat16)
    a = jax.lax.dot_general(
        xb, w1_ref[0].astype(jnp.bfloat16), (((1,), (0,)), ((), ())),
        preferred_element_type=jnp.float32)
    b = jax.lax.dot_general(
        xb, w3_ref[0].astype(jnp.bfloat16), (((1,), (0,)), ((), ())),
        preferred_element_type=jnp.float32)
    h = (a * (1.0 / (1.0 + jnp.exp(-a))) * b).astype(jnp.bfloat16)
    y = jax.lax.dot_general(
        h, w2_ref[0].astype(jnp.bfloat16), (((1,), (0,)), ((), ())),
        preferred_element_type=jnp.float32)

    rows = m * BM + jax.lax.broadcasted_iota(jnp.int32, (BM, 1), 0)
    y = jnp.where((rows >= lo) & (rows < hi), y, 0.0)
    if has_prev:
        @pl.when(first == 1)
        def _():
            out_ref[...] = prev_ref[...] + y

        @pl.when(first == 0)
        def _():
            out_ref[...] += y
    else:
        @pl.when(first == 1)
        def _():
            out_ref[...] = y

        @pl.when(first == 0)
        def _():
            out_ref[...] += y


def _gffn_half(xs, w1, w3, w2, meta, n_items, nh, prev=None):
    R = xs.shape[0]
    in_specs = [
        pl.BlockSpec((BM, DIM), lambda i, meta: (meta[1, i], 0)),
        pl.BlockSpec((1, DIM, HID2), lambda i, meta: (meta[0, i], 0, nh)),
        pl.BlockSpec((1, DIM, HID2), lambda i, meta: (meta[0, i], 0, nh)),
        pl.BlockSpec((1, HID2, DIM), lambda i, meta: (meta[0, i], nh, 0)),
    ]
    args = [meta, xs, w1, w3, w2]
    kwargs = {}
    if prev is not None:
        in_specs.append(pl.BlockSpec((BM, DIM), lambda i, meta: (meta[1, i], 0)))
        args.append(prev)
        kwargs["input_output_aliases"] = {5: 0}
    grid_spec = pltpu.PrefetchScalarGridSpec(
        num_scalar_prefetch=1,
        grid=(n_items,),
        in_specs=in_specs,
        out_specs=pl.BlockSpec((BM, DIM), lambda i, meta: (meta[1, i], 0)),
    )
    return pl.pallas_call(
        functools.partial(_gffn_half_body, prev is not None),
        grid_spec=grid_spec,
        out_shape=jax.ShapeDtypeStruct((R, DIM), jnp.float32),
        **kwargs,
    )(*args)


def _expert_meta(counts, n_rows, n_items):
    ends = jnp.cumsum(counts)
    starts = ends - counts
    f = starts // BM
    l = (ends - 1) // BM
    tiles = jnp.where(counts > 0, l - f + 1, 0)
    c_incl = jnp.cumsum(tiles)
    c_excl = c_incl - tiles
    n_real = c_incl[-1]
    i = jnp.arange(n_items, dtype=jnp.int32)
    e_of = jnp.sum(c_incl[None, :] <= i[:, None], axis=1)
    e_of = jnp.clip(e_of, 0, counts.shape[0] - 1).astype(jnp.int32)
    m_of = (f[e_of] + (i - c_excl[e_of])).astype(jnp.int32)
    valid = i < n_real
    last_m = (n_rows // BM) - 1
    m_of = jnp.where(valid, m_of, last_m)
    lo = jnp.where(valid, jnp.maximum(starts[e_of], m_of * BM), n_rows)
    hi = jnp.where(valid, jnp.minimum(ends[e_of], (m_of + 1) * BM), n_rows)
    first = jnp.concatenate(
        [jnp.ones((1,), jnp.int32),
         (m_of[1:] != m_of[:-1]).astype(jnp.int32)])
    first = jnp.where(valid, first, 0)
    return jnp.stack([e_of, m_of, first,
                      lo.astype(jnp.int32), hi.astype(jnp.int32)]).astype(jnp.int32)


def _shared_body(x_ref, w1_ref, w3_ref, w2_ref, out_ref):
    xb = x_ref[...]
    a = jax.lax.dot_general(
        xb, w1_ref[...], (((1,), (0,)), ((), ())),
        preferred_element_type=jnp.float32)
    b = jax.lax.dot_general(
        xb, w3_ref[...], (((1,), (0,)), ((), ())),
        preferred_element_type=jnp.float32)
    h = (a * (1.0 / (1.0 + jnp.exp(-a))) * b).astype(jnp.bfloat16)
    out_ref[...] = jax.lax.dot_general(
        h, w2_ref[...], (((1,), (0,)), ((), ())),
        preferred_element_type=jnp.float32)


def _shared_ffn(xtb, sw1b, sw3b, sw2b):
    T = xtb.shape[0]
    return pl.pallas_call(
        _shared_body,
        grid=(T // BMS,),
        in_specs=[
            pl.BlockSpec((BMS, DIM), lambda i: (i, 0)),
            pl.BlockSpec((DIM, HIDDEN), lambda i: (0, 0)),
            pl.BlockSpec((DIM, HIDDEN), lambda i: (0, 0)),
            pl.BlockSpec((HIDDEN, DIM), lambda i: (0, 0)),
        ],
        out_specs=pl.BlockSpec((BMS, DIM), lambda i: (i, 0)),
        out_shape=jax.ShapeDtypeStruct((T, DIM), jnp.float32),
    )(xtb, sw1b, sw3b, sw2b)


def _combine_body(has_prev, *refs):
    if has_prev:
        sh_ref, g0_ref, g1_ref, w0_ref, w1_ref, _prev, out_ref = refs
    else:
        sh_ref, g0_ref, g1_ref, w0_ref, w1_ref, out_ref = refs
    out_ref[...] = (sh_ref[...] + w0_ref[...] * g0_ref[...]
                    + w1_ref[...] * g1_ref[...])


def _combine_half(shared, g0, g1, w0, w1, half, prev=None):
    T = shared.shape[0]
    T2 = T // 2
    CB = 512
    nb = T2 // CB
    offb = half * nb
    in_specs = [
        pl.BlockSpec((CB, DIM), lambda i, offb=offb: (i + offb, 0)),
        pl.BlockSpec((CB, DIM), lambda i: (i, 0)),
        pl.BlockSpec((CB, DIM), lambda i: (i, 0)),
        pl.BlockSpec((CB, 1), lambda i, offb=offb: (i + offb, 0)),
        pl.BlockSpec((CB, 1), lambda i, offb=offb: (i + offb, 0)),
    ]
    args = [shared, g0, g1, w0, w1]
    kwargs = {}
    if prev is not None:
        in_specs.append(
            pl.BlockSpec((CB, DIM), lambda i, offb=offb: (i + offb, 0)))
        args.append(prev)
        kwargs["input_output_aliases"] = {5: 0}
    return pl.pallas_call(
        functools.partial(_combine_body, prev is not None),
        grid=(nb,),
        in_specs=in_specs,
        out_specs=pl.BlockSpec((CB, DIM), lambda i, offb=offb: (i + offb, 0)),
        out_shape=jax.ShapeDtypeStruct((T, DIM), jnp.float32),
        **kwargs,
    )(*args)


def kernel(x, w_router, w1, w2, w3, sw1, sw2, sw3):
    bs, slen, dim = x.shape
    T = bs * slen
    R = T * TOPK
    xt = x.reshape(T, dim)

    sel, wts, rank, counts, base, xtb = _router(xt, w_router)

    xs, pos0, pos1 = _make_dispatch(T)(
        xt, sel[:, 0], sel[:, 1], rank[:, 0], rank[:, 1],
        jnp.pad(base[0], (0, L - E)))

    n_items = R // BM + E - 1
    meta = _expert_meta(counts[0], R, n_items)
    part = _gffn_half(xs, w1, w3, w2, meta, n_items, 0)
    ys = _gffn_half(xs, w1, w3, w2, meta, n_items, 1, prev=part)

    shared = _shared_ffn(xtb, sw1.astype(jnp.bfloat16),
                         sw3.astype(jnp.bfloat16), sw2.astype(jnp.bfloat16))

    T2 = T // 2
    gat = _make_gather(T2)
    g0a, g1a = gat(ys, pos0[:T2], pos1[:T2])
    g0b, g1b = gat(ys, pos0[T2:], pos1[T2:])
    w0 = wts[:, 0:1]
    w1 = wts[:, 1:2]
    out_a = _combine_half(shared, g0a, g1a, w0, w1, 0)
    out = _combine_half(shared, g0b, g1b, w0, w1, 1, prev=out_a)
    return out.reshape(bs, slen, dim)


# speedup vs baseline: 2.2565x; 1.0116x over previous
import functools

import jax
import jax.numpy as jnp
from jax import lax
from jax.experimental import pallas as pl
from jax.experimental.pallas import tpu as pltpu
from jax.experimental.pallas import tpu_sc as plsc

E = 8
TOPK = 2
DIM = 2048
HIDDEN = 2048
HID2 = HIDDEN // 2
D32 = DIM // 2

BM = 128
BMS = 256
RBM = 512
NWORKERS = 32
L = 16


def _router_body(x_ref, wr_ref, sel_ref, wts_ref, rank_ref, cnt_ref,
                 base_ref, xtb_ref, cnt_scratch):
    xtb_ref[...] = x_ref[...].astype(jnp.bfloat16)
    i = pl.program_id(0)

    @pl.when(i == 0)
    def _():
        cnt_scratch[...] = jnp.zeros_like(cnt_scratch)

    xb = x_ref[...]
    wr = wr_ref[...]
    logits = jax.lax.dot_general(
        xb, wr, (((1,), (1,)), ((), ())),
        preferred_element_type=jnp.float32,
        precision=jax.lax.Precision.DEFAULT)
    iota = jax.lax.broadcasted_iota(jnp.int32, logits.shape, 1)
    m1 = jnp.max(logits, axis=1, keepdims=True)
    i1 = jnp.min(jnp.where(logits == m1, iota, E), axis=1, keepdims=True)
    masked = jnp.where(iota == i1, -jnp.inf, logits)
    m2 = jnp.max(masked, axis=1, keepdims=True)
    i2 = jnp.min(jnp.where(masked == m2, iota, E), axis=1, keepdims=True)
    w0 = 1.0 / (1.0 + jnp.exp(m2 - m1))
    sel_ref[...] = jnp.concatenate([i1, i2], axis=1)
    wts_ref[...] = jnp.concatenate([w0, 1.0 - w0], axis=1)

    oh0 = (iota == i1).astype(jnp.float32)
    oh1 = (iota == i2).astype(jnp.float32)
    r_iota = jax.lax.broadcasted_iota(jnp.int32, (RBM, RBM), 0)
    c_iota = jax.lax.broadcasted_iota(jnp.int32, (RBM, RBM), 1)
    strict_tril = (r_iota > c_iota).astype(jnp.float32)
    excl0 = jax.lax.dot_general(
        strict_tril, oh0, (((1,), (0,)), ((), ())),
        preferred_element_type=jnp.float32)
    excl1 = jax.lax.dot_general(
        strict_tril, oh1, (((1,), (0,)), ((), ())),
        preferred_element_type=jnp.float32)
    tot0 = jnp.sum(oh0, axis=0, keepdims=True)
    tot1 = jnp.sum(oh1, axis=0, keepdims=True)
    cnt = cnt_scratch[...]
    rank0 = jnp.sum(oh0 * (excl0 + cnt), axis=1, keepdims=True)
    rank1 = jnp.sum(oh1 * (excl1 + cnt + tot0), axis=1, keepdims=True)
    rank_ref[...] = jnp.concatenate([rank0, rank1], axis=1).astype(jnp.int32)
    new_cnt = cnt + tot0 + tot1
    cnt_scratch[...] = new_cnt
    cnt_ref[...] = new_cnt.astype(jnp.int32)
    e_r = jax.lax.broadcasted_iota(jnp.int32, (E, E), 0)
    e_c = jax.lax.broadcasted_iota(jnp.int32, (E, E), 1)
    strict = (e_r < e_c).astype(jnp.float32)
    base_ref[...] = jax.lax.dot_general(
        new_cnt, strict, (((1,), (0,)), ((), ())),
        preferred_element_type=jnp.float32,
        precision=jax.lax.Precision.HIGHEST).astype(jnp.int32)


def _router(xt, w_router):
    T = xt.shape[0]
    return pl.pallas_call(
        _router_body,
        grid=(T // RBM,),
        in_specs=[
            pl.BlockSpec((RBM, DIM), lambda i: (i, 0)),
            pl.BlockSpec((E, DIM), lambda i: (0, 0)),
        ],
        out_specs=[
            pl.BlockSpec((RBM, TOPK), lambda i: (i, 0)),
            pl.BlockSpec((RBM, TOPK), lambda i: (i, 0)),
            pl.BlockSpec((RBM, TOPK), lambda i: (i, 0)),
            pl.BlockSpec((1, E), lambda i: (0, 0)),
            pl.BlockSpec((1, E), lambda i: (0, 0)),
            pl.BlockSpec((RBM, DIM), lambda i: (i, 0)),
        ],
        out_shape=[
            jax.ShapeDtypeStruct((T, TOPK), jnp.int32),
            jax.ShapeDtypeStruct((T, TOPK), jnp.float32),
            jax.ShapeDtypeStruct((T, TOPK), jnp.int32),
            jax.ShapeDtypeStruct((1, E), jnp.int32),
            jax.ShapeDtypeStruct((1, E), jnp.int32),
            jax.ShapeDtypeStruct((T, DIM), jnp.bfloat16),
        ],
        scratch_shapes=[pltpu.VMEM((1, E), jnp.float32)],
    )(xt, w_router)


def _make_dispatch(T):
    t_per_w = T // NWORKERS
    n_chunks = t_per_w // L
    mesh = plsc.VectorSubcoreMesh(core_axis_name="c", subcore_axis_name="s")

    @functools.partial(
        pl.kernel, mesh=mesh,
        out_type=[
            jax.ShapeDtypeStruct((T * TOPK, DIM), jnp.float32),
            jax.ShapeDtypeStruct((T,), jnp.int32),
            jax.ShapeDtypeStruct((T,), jnp.int32),
        ],
        scratch_types=[
            pltpu.VMEM((L,), jnp.int32),
            pltpu.VMEM((t_per_w,), jnp.int32),
            pltpu.VMEM((t_per_w,), jnp.int32),
            pltpu.VMEM((t_per_w,), jnp.int32),
            pltpu.VMEM((t_per_w,), jnp.int32),
            pltpu.VMEM((t_per_w,), jnp.int32),
            pltpu.VMEM((t_per_w,), jnp.int32),
            pltpu.VMEM((L, DIM), jnp.float32),
            pltpu.SemaphoreType.DMA,
        ],
        compiler_params=pltpu.CompilerParams(needs_layout_passes=False),
    )
    def dispatch(xt_hbm, sel0_hbm, sel1_hbm, rank0_hbm, rank1_hbm, base_hbm,
                 xs_hbm, pos0_hbm, pos1_hbm,
                 base_v, sel0_v, sel1_v, rank0_v, rank1_v,
                 pos0_v, pos1_v, buf, sem):
        wid = lax.axis_index("s") * 2 + lax.axis_index("c")
        tbase = wid * t_per_w
        pltpu.sync_copy(base_hbm, base_v)
        pltpu.sync_copy(sel0_hbm.at[pl.ds(tbase, t_per_w)], sel0_v)
        pltpu.sync_copy(sel1_hbm.at[pl.ds(tbase, t_per_w)], sel1_v)
        pltpu.sync_copy(rank0_hbm.at[pl.ds(tbase, t_per_w)], rank0_v)
        pltpu.sync_copy(rank1_hbm.at[pl.ds(tbase, t_per_w)], rank1_v)
        for c in range(n_chunks):
            v0 = sel0_v[pl.ds(c * L, L)]
            v1 = sel1_v[pl.ds(c * L, L)]
            p0 = plsc.load_gather(base_v, [v0]) + rank0_v[pl.ds(c * L, L)]
            p1 = plsc.load_gather(base_v, [v1]) + rank1_v[pl.ds(c * L, L)]
            pos0_v[pl.ds(c * L, L)] = p0
            pos1_v[pl.ds(c * L, L)] = p1
            pltpu.sync_copy(xt_hbm.at[pl.ds(tbase + c * L, L)], buf)
            pltpu.async_copy(buf, xs_hbm.at[p0], sem).wait()
            pltpu.async_copy(buf, xs_hbm.at[p1], sem).wait()
        pltpu.sync_copy(pos0_v, pos0_hbm.at[pl.ds(tbase, t_per_w)])
        pltpu.sync_copy(pos1_v, pos1_hbm.at[pl.ds(tbase, t_per_w)])

    return dispatch


def _make_gather(T):
    t_per_w = T // NWORKERS
    n_chunks = t_per_w // L
    nbuf = 3
    mesh = plsc.VectorSubcoreMesh(core_axis_name="c", subcore_axis_name="s")

    @functools.partial(
        pl.kernel, mesh=mesh,
        out_type=[
            jax.ShapeDtypeStruct((T, DIM), jnp.float32),
            jax.ShapeDtypeStruct((T, DIM), jnp.float32),
        ],
        scratch_types=[
            pltpu.VMEM((t_per_w,), jnp.int32),
            pltpu.VMEM((t_per_w,), jnp.int32),
            pltpu.VMEM((L, DIM), jnp.float32),
            pltpu.VMEM((L, DIM), jnp.float32),
            pltpu.VMEM((L, DIM), jnp.float32),
            pltpu.SemaphoreType.DMA,
            pltpu.SemaphoreType.DMA,
            pltpu.SemaphoreType.DMA,
        ],
        compiler_params=pltpu.CompilerParams(needs_layout_passes=False),
    )
    def gather(ys_hbm, pos0_hbm, pos1_hbm, g0_hbm, g1_hbm,
               pos0_v, pos1_v, bufa, bufb, bufc, sema, semb, semc):
        wid = lax.axis_index("s") * 2 + lax.axis_index("c")
        tbase = wid * t_per_w
        pltpu.sync_copy(pos0_hbm.at[pl.ds(tbase, t_per_w)], pos0_v)
        pltpu.sync_copy(pos1_hbm.at[pl.ds(tbase, t_per_w)], pos1_v)
        bufs = (bufa, bufb, bufc)
        sems = (sema, semb, semc)
        pos_vs = (pos0_v, pos1_v)
        g_hbms = (g0_hbm, g1_hbm)
        n_tr = 2 * n_chunks

        def start(i):
            c, col = divmod(i, 2)
            q = pos_vs[col][pl.ds(c * L, L)]
            return pltpu.async_copy(ys_hbm.at[q], bufs[i % nbuf],
                                    sems[i % nbuf])

        pend = [None] * n_tr
        for i in range(min(nbuf, n_tr)):
            pend[i] = start(i)
        for i in range(n_tr):
            c, col = divmod(i, 2)
            pend[i].wait()
            pltpu.sync_copy(bufs[i % nbuf],
                            g_hbms[col].at[pl.ds(tbase + c * L, L)])
            if i + nbuf < n_tr:
                pend[i + nbuf] = start(i + nbuf)

    return gather


def _gffn_half_body(has_prev, bm, meta_ref, *refs):
    if has_prev:
        x_ref, w1_ref, w3_ref, w2_ref, prev_ref, out_ref = refs
    else:
        x_ref, w1_ref, w3_ref, w2_ref, out_ref = refs
    i = pl.program_id(0)
    first = meta_ref[2, i]
    lo = meta_ref[3, i]
    hi = meta_ref[4, i]
    m = meta_ref[1, i]

    xb = x_ref[...].astype(jnp.bfloat16)
    a = jax.lax.dot_general(
        xb, w1_ref[0].astype(jnp.bfloat16), (((1,), (0,)), ((), ())),
        preferred_element_type=jnp.float32)
    b = jax.lax.dot_general(
        xb, w3_ref[0].astype(jnp.bfloat16), (((1,), (0,)), ((), ())),
        preferred_element_type=jnp.float32)
    h = (a * (1.0 / (1.0 + jnp.exp(-a))) * b).astype(jnp.bfloat16)
    y = jax.lax.dot_general(
        h, w2_ref[0].astype(jnp.bfloat16), (((1,), (0,)), ((), ())),
        preferred_element_type=jnp.float32)

    rows = m * bm + jax.lax.broadcasted_iota(jnp.int32, (bm, 1), 0)
    y = jnp.where((rows >= lo) & (rows < hi), y, 0.0)
    if has_prev:
        @pl.when(first == 1)
        def _():
            out_ref[...] = prev_ref[...] + y

        @pl.when(first == 0)
        def _():
            out_ref[...] += y
    else:
        @pl.when(first == 1)
        def _():
            out_ref[...] = y

        @pl.when(first == 0)
        def _():
            out_ref[...] += y


def _gffn_half(xs, w1, w3, w2, meta, n_items, nh, bm, prev=None):
    R = xs.shape[0]
    in_specs = [
        pl.BlockSpec((bm, DIM), lambda i, meta: (meta[1, i], 0)),
        pl.BlockSpec((1, DIM, HID2), lambda i, meta: (meta[0, i], 0, nh)),
        pl.BlockSpec((1, DIM, HID2), lambda i, meta: (meta[0, i], 0, nh)),
        pl.BlockSpec((1, HID2, DIM), lambda i, meta: (meta[0, i], nh, 0)),
    ]
    args = [meta, xs, w1, w3, w2]
    kwargs = {}
    if prev is not None:
        in_specs.append(pl.BlockSpec((bm, DIM), lambda i, meta: (meta[1, i], 0)))
        args.append(prev)
        kwargs["input_output_aliases"] = {5: 0}
    grid_spec = pltpu.PrefetchScalarGridSpec(
        num_scalar_prefetch=1,
        grid=(n_items,),
        in_specs=in_specs,
        out_specs=pl.BlockSpec((bm, DIM), lambda i, meta: (meta[1, i], 0)),
    )
    return pl.pallas_call(
        functools.partial(_gffn_half_body, prev is not None, bm),
        grid_spec=grid_spec,
        out_shape=jax.ShapeDtypeStruct((R, DIM), jnp.float32),
        **kwargs,
    )(*args)


def _expert_meta(counts, n_rows, n_items, bm):
    BM = bm
    ends = jnp.cumsum(counts)
    starts = ends - counts
    f = starts // BM
    l = (ends - 1) // BM
    tiles = jnp.where(counts > 0, l - f + 1, 0)
    c_incl = jnp.cumsum(tiles)
    c_excl = c_incl - tiles
    n_real = c_incl[-1]
    i = jnp.arange(n_items, dtype=jnp.int32)
    e_of = jnp.sum(c_incl[None, :] <= i[:, None], axis=1)
    e_of = jnp.clip(e_of, 0, counts.shape[0] - 1).astype(jnp.int32)
    m_of = (f[e_of] + (i - c_excl[e_of])).astype(jnp.int32)
    valid = i < n_real
    last_m = (n_rows // BM) - 1
    m_of = jnp.where(valid, m_of, last_m)
    lo = jnp.where(valid, jnp.maximum(starts[e_of], m_of * BM), n_rows)
    hi = jnp.where(valid, jnp.minimum(ends[e_of], (m_of + 1) * BM), n_rows)
    first = jnp.concatenate(
        [jnp.ones((1,), jnp.int32),
         (m_of[1:] != m_of[:-1]).astype(jnp.int32)])
    first = jnp.where(valid, first, 0)
    return jnp.stack([e_of, m_of, first,
                      lo.astype(jnp.int32), hi.astype(jnp.int32)]).astype(jnp.int32)


def _shared_body(x_ref, w1_ref, w3_ref, w2_ref, out_ref):
    xb = x_ref[...]
    a = jax.lax.dot_general(
        xb, w1_ref[...], (((1,), (0,)), ((), ())),
        preferred_element_type=jnp.float32)
    b = jax.lax.dot_general(
        xb, w3_ref[...], (((1,), (0,)), ((), ())),
        preferred_element_type=jnp.float32)
    h = (a * (1.0 / (1.0 + jnp.exp(-a))) * b).astype(jnp.bfloat16)
    out_ref[...] = jax.lax.dot_general(
        h, w2_ref[...], (((1,), (0,)), ((), ())),
        preferred_element_type=jnp.float32)


def _shared_ffn(xtb, sw1b, sw3b, sw2b):
    T = xtb.shape[0]
    return pl.pallas_call(
        _shared_body,
        grid=(T // BMS,),
        in_specs=[
            pl.BlockSpec((BMS, DIM), lambda i: (i, 0)),
            pl.BlockSpec((DIM, HIDDEN), lambda i: (0, 0)),
            pl.BlockSpec((DIM, HIDDEN), lambda i: (0, 0)),
            pl.BlockSpec((HIDDEN, DIM), lambda i: (0, 0)),
        ],
        out_specs=pl.BlockSpec((BMS, DIM), lambda i: (i, 0)),
        out_shape=jax.ShapeDtypeStruct((T, DIM), jnp.float32),
    )(xtb, sw1b, sw3b, sw2b)


def _combine_body(has_prev, *refs):
    if has_prev:
        sh_ref, g0_ref, g1_ref, w0_ref, w1_ref, _prev, out_ref = refs
    else:
        sh_ref, g0_ref, g1_ref, w0_ref, w1_ref, out_ref = refs
    out_ref[...] = (sh_ref[...] + w0_ref[...] * g0_ref[...]
                    + w1_ref[...] * g1_ref[...])


def _combine_half(shared, g0, g1, w0, w1, half, prev=None):
    T = shared.shape[0]
    T2 = T // 2
    CB = 512
    nb = T2 // CB
    offb = half * nb
    in_specs = [
        pl.BlockSpec((CB, DIM), lambda i, offb=offb: (i + offb, 0)),
        pl.BlockSpec((CB, DIM), lambda i: (i, 0)),
        pl.BlockSpec((CB, DIM), lambda i: (i, 0)),
        pl.BlockSpec((CB, 1), lambda i, offb=offb: (i + offb, 0)),
        pl.BlockSpec((CB, 1), lambda i, offb=offb: (i + offb, 0)),
    ]
    args = [shared, g0, g1, w0, w1]
    kwargs = {}
    if prev is not None:
        in_specs.append(
            pl.BlockSpec((CB, DIM), lambda i, offb=offb: (i + offb, 0)))
        args.append(prev)
        kwargs["input_output_aliases"] = {5: 0}
    return pl.pallas_call(
        functools.partial(_combine_body, prev is not None),
        grid=(nb,),
        in_specs=in_specs,
        out_specs=pl.BlockSpec((CB, DIM), lambda i, offb=offb: (i + offb, 0)),
        out_shape=jax.ShapeDtypeStruct((T, DIM), jnp.float32),
        **kwargs,
    )(*args)


def kernel(x, w_router, w1, w2, w3, sw1, sw2, sw3):
    bs, slen, dim = x.shape
    T = bs * slen
    R = T * TOPK
    xt = x.reshape(T, dim)

    sel, wts, rank, counts, base, xtb = _router(xt, w_router)

    xs, pos0, pos1 = _make_dispatch(T)(
        xt, sel[:, 0], sel[:, 1], rank[:, 0], rank[:, 1],
        jnp.pad(base[0], (0, L - E)))

    BM1 = 256
    n_items1 = R // BM1 + E - 1
    meta1 = _expert_meta(counts[0], R, n_items1, BM1)
    n_items2 = R // BM + E - 1
    meta2 = _expert_meta(counts[0], R, n_items2, BM)
    part = _gffn_half(xs, w1, w3, w2, meta1, n_items1, 0, BM1)
    ys = _gffn_half(xs, w1, w3, w2, meta2, n_items2, 1, BM, prev=part)

    shared = _shared_ffn(xtb, sw1.astype(jnp.bfloat16),
                         sw3.astype(jnp.bfloat16), sw2.astype(jnp.bfloat16))

    T2 = T // 2
    gat = _make_gather(T2)
    g0a, g1a = gat(ys, pos0[:T2], pos1[:T2])
    g0b, g1b = gat(ys, pos0[T2:], pos1[T2:])
    w0 = wts[:, 0:1]
    w1 = wts[:, 1:2]
    out_a = _combine_half(shared, g0a, g1a, w0, w1, 0)
    out = _combine_half(shared, g0b, g1b, w0, w1, 1, prev=out_a)
    return out.reshape(bs, slen, dim)
